# baseline (speedup 1.0000x reference)
import functools

import jax
import jax.numpy as jnp
from jax import lax
from jax.experimental import pallas as pl
from jax.experimental.pallas import tpu as pltpu
from jax.experimental.pallas import tpu_sc as plsc

N = 10000
E = 320000
D = 128

_INFO = plsc.get_sparse_core_info()
NC = _INFO.num_cores
NS = _INFO.num_subcores
NW = NC * NS

NP = 10240
ROWS_PER_TILE = NP // NS
CH = 128
NCH = 80
NB = 2
IB = 40
NIB = NCH // IB
EPT = NCH * CH
EPAD = EPT * NW

_MESH = plsc.VectorSubcoreMesh(core_axis_name="c", subcore_axis_name="s")
_SC_PARAMS = pltpu.CompilerParams(needs_layout_passes=False)


@functools.partial(
    pl.kernel,
    out_type=jax.ShapeDtypeStruct((NW, NP), jnp.float32),
    mesh=_MESH,
    scratch_types=[
        pltpu.VMEM((NCH, CH), jnp.int32),
        pltpu.VMEM((NP,), jnp.float32),
    ],
    compiler_params=_SC_PARAMS,
)
def _sc_degree(edges_hbm, out_hbm, dst_v, hist_v):
    c = lax.axis_index("c")
    s = lax.axis_index("s")
    wid = s * NC + c

    def zero(i, _):
        hist_v[pl.ds(i * 16, 16)] = jnp.zeros((16,), jnp.float32)
        return _

    lax.fori_loop(0, NP // 16, zero, None)

    pltpu.sync_copy(edges_hbm.at[1, wid], dst_v)

    ones = jnp.ones((16,), jnp.float32)

    def row(r, _):
        def col(k, __):
            idx = dst_v[r, pl.ds(k * 16, 16)]
            plsc.addupdate_scatter(hist_v, [idx], ones)
            return __

        return lax.fori_loop(0, CH // 16, col, _)

    lax.fori_loop(0, NCH, row, None)

    pltpu.sync_copy(hist_v, out_hbm.at[wid])


@functools.partial(
    pl.kernel,
    out_type=jax.ShapeDtypeStruct((NC, NP, D), jnp.float32),
    mesh=_MESH,
    scratch_types=[
        pltpu.VMEM((IB, CH), jnp.int32),
        pltpu.VMEM((IB, CH), jnp.int32),
        [pltpu.VMEM((CH, D), jnp.float32)] * NB,
        pltpu.VMEM_SHARED((NP, D), jnp.float32),
        [pltpu.SemaphoreType.DMA] * NB,
        [pltpu.SemaphoreType.DMA] * NB,
    ],
    compiler_params=_SC_PARAMS,
)
def _sc_hop(x_hbm, edges_hbm, zeros_hbm, out_hbm, sidx, didx, rows, acc_sh, gsem, ssem):
    c = lax.axis_index("c")
    s = lax.axis_index("s")
    wid = s * NC + c

    pltpu.sync_copy(zeros_hbm, acc_sh.at[pl.ds(s * ROWS_PER_TILE, ROWS_PER_TILE)])
    pltpu.sync_copy(edges_hbm.at[0, wid, pl.ds(0, IB)], sidx)
    pltpu.sync_copy(edges_hbm.at[1, wid, pl.ds(0, IB)], didx)
    pltpu.async_copy(x_hbm.at[sidx.at[0]], rows[0], gsem[0])

    plsc.subcore_barrier()

    for kb in range(NIB):
        if kb > 0:
            pltpu.sync_copy(edges_hbm.at[0, wid, pl.ds(kb * IB, IB)], sidx)
            pltpu.sync_copy(edges_hbm.at[1, wid, pl.ds(kb * IB, IB)], didx)
            pltpu.async_copy(x_hbm.at[sidx.at[0]], rows[0], gsem[0])

        def pair(t, _):
            for b in range(NB):
                jj = t * NB + b
                bp = 1 - b

                @pl.when(jj >= 1)
                def _recycle():
                    pltpu.make_async_copy(
                        rows[bp], acc_sh.at[didx.at[jj - 1]], ssem[bp]
                    ).wait()

                @pl.when(jj + 1 < IB)
                def _prefetch():
                    pltpu.async_copy(x_hbm.at[sidx.at[jj + 1]], rows[bp], gsem[bp])

                pltpu.make_async_copy(x_hbm.at[sidx.at[jj]], rows[b], gsem[b]).wait()
                pltpu.async_copy(rows[b], acc_sh.at[didx.at[jj]], ssem[b], add=True)
            return _

        lax.fori_loop(0, IB // NB, pair, None)

        pltpu.make_async_copy(rows[1], acc_sh.at[didx.at[IB - 1]], ssem[1]).wait()

    plsc.subcore_barrier()

    pltpu.sync_copy(
        acc_sh.at[pl.ds(s * ROWS_PER_TILE, ROWS_PER_TILE)],
        out_hbm.at[c, pl.ds(s * ROWS_PER_TILE, ROWS_PER_TILE)],
    )


_RB = 2048


def _tc_prep_body(deg_ref, feat_ref, z0_ref, norm_ref, norm2_ref):
    d = jnp.sum(deg_ref[...], axis=0)
    d = jnp.maximum(d, 1.0)
    nrm = lax.rsqrt(d)[:, None]
    norm_ref[...] = jnp.broadcast_to(nrm, (_RB, D))
    norm2_ref[...] = jnp.broadcast_to((1.0 / d)[:, None], (_RB, D))
    z0_ref[...] = feat_ref[...] * nrm


_tc_prep = pl.pallas_call(
    _tc_prep_body,
    grid=(NP // _RB,),
    in_specs=[
        pl.BlockSpec((NW, _RB), lambda i: (0, i)),
        pl.BlockSpec((_RB, D), lambda i: (i, 0)),
    ],
    out_specs=[
        pl.BlockSpec((_RB, D), lambda i: (i, 0)),
        pl.BlockSpec((_RB, D), lambda i: (i, 0)),
        pl.BlockSpec((_RB, D), lambda i: (i, 0)),
    ],
    out_shape=[
        jax.ShapeDtypeStruct((NP, D), jnp.float32),
        jax.ShapeDtypeStruct((NP, D), jnp.float32),
        jax.ShapeDtypeStruct((NP, D), jnp.float32),
    ],
)


def _tc_scale_body(acc_ref, norm2_ref, z1_ref):
    z1_ref[...] = (acc_ref[0] + acc_ref[1]) * norm2_ref[...]


_tc_scale = pl.pallas_call(
    _tc_scale_body,
    grid=(NP // _RB,),
    in_specs=[
        pl.BlockSpec((NC, _RB, D), lambda i: (0, i, 0)),
        pl.BlockSpec((_RB, D), lambda i: (i, 0)),
    ],
    out_specs=pl.BlockSpec((_RB, D), lambda i: (i, 0)),
    out_shape=jax.ShapeDtypeStruct((NP, D), jnp.float32),
)


def _tc_final_body(acc_ref, norm_ref, w_ref, b_ref, out_ref):
    h = (acc_ref[0] + acc_ref[1]) * norm_ref[...]
    out_ref[...] = (
        jnp.dot(h, w_ref[...], preferred_element_type=jnp.float32) + b_ref[...]
    )


_tc_final = pl.pallas_call(
    _tc_final_body,
    grid=(NP // _RB,),
    in_specs=[
        pl.BlockSpec((NC, _RB, D), lambda i: (0, i, 0)),
        pl.BlockSpec((_RB, D), lambda i: (i, 0)),
        pl.BlockSpec((D, D), lambda i: (0, 0)),
        pl.BlockSpec((1, D), lambda i: (0, 0)),
    ],
    out_specs=pl.BlockSpec((_RB, D), lambda i: (i, 0)),
    out_shape=jax.ShapeDtypeStruct((NP, D), jnp.float32),
)


def kernel(feat, edge_index, weight, bias):
    feat_p = jnp.pad(feat, ((0, NP - N), (0, 0)))
    edges_p = jnp.pad(edge_index, ((0, 0), (0, EPAD - E)), constant_values=NP - 1)
    edges_r = edges_p.reshape(2, NW, NCH, CH)

    zeros_rows = jnp.zeros((ROWS_PER_TILE, D), jnp.float32)

    deg_parts = _sc_degree(edges_r)
    z0, norm_m, norm2_m = _tc_prep(deg_parts, feat_p)
    acc_a = _sc_hop(z0, edges_r, zeros_rows)
    z1 = _tc_scale(acc_a, norm2_m)
    acc_b = _sc_hop(z1, edges_r, zeros_rows)
    out = _tc_final(acc_b, norm_m, weight, bias.reshape(1, D))
    return out[:N]


# speedup vs baseline: 1.0167x; 1.0167x over previous
import functools

import jax
import jax.numpy as jnp
from jax import lax
from jax.experimental import pallas as pl
from jax.experimental.pallas import tpu as pltpu
from jax.experimental.pallas import tpu_sc as plsc

N = 10000
E = 320000
D = 128

_INFO = plsc.get_sparse_core_info()
NC = _INFO.num_cores
NS = _INFO.num_subcores
NW = NC * NS

NP = 10240
ROWS_PER_TILE = NP // NS
CH = 128
NCH = 80
NB = 2
IB = 40
NIB = NCH // IB
EPT = NCH * CH
EPAD = EPT * NW

_MESH = plsc.VectorSubcoreMesh(core_axis_name="c", subcore_axis_name="s")
_MESH1 = plsc.VectorSubcoreMesh(
    core_axis_name="c", subcore_axis_name="s", num_cores=1
)
_SC_PARAMS = pltpu.CompilerParams(needs_layout_passes=False)

NCH1 = 2 * NCH
NIB1 = NCH1 // IB


@functools.partial(
    pl.kernel,
    out_type=jax.ShapeDtypeStruct((NW, NP), jnp.float32),
    mesh=_MESH,
    scratch_types=[
        pltpu.VMEM((NCH, CH), jnp.int32),
        pltpu.VMEM((NP,), jnp.float32),
    ],
    compiler_params=_SC_PARAMS,
)
def _sc_degree(edges_hbm, out_hbm, dst_v, hist_v):
    c = lax.axis_index("c")
    s = lax.axis_index("s")
    wid = s * NC + c

    def zero(i, _):
        hist_v[pl.ds(i * 16, 16)] = jnp.zeros((16,), jnp.float32)
        return _

    lax.fori_loop(0, NP // 16, zero, None)

    pltpu.sync_copy(edges_hbm.at[1, wid], dst_v)

    ones = jnp.ones((16,), jnp.float32)

    def row(r, _):
        def col(k, __):
            idx = dst_v[r, pl.ds(k * 16, 16)]
            plsc.addupdate_scatter(hist_v, [idx], ones)
            return __

        return lax.fori_loop(0, CH // 16, col, _)

    lax.fori_loop(0, NCH, row, None)

    pltpu.sync_copy(hist_v, out_hbm.at[wid])


@functools.partial(
    pl.kernel,
    out_type=jax.ShapeDtypeStruct((NP, D), jnp.float32),
    mesh=_MESH1,
    scratch_types=[
        pltpu.VMEM((IB, CH), jnp.int32),
        pltpu.VMEM((IB, CH), jnp.int32),
        [pltpu.VMEM((CH, D), jnp.float32)] * NB,
        pltpu.VMEM_SHARED((NP, D), jnp.float32),
        [pltpu.SemaphoreType.DMA] * NB,
        [pltpu.SemaphoreType.DMA] * NB,
    ],
    compiler_params=_SC_PARAMS,
)
def _sc_hop(x_hbm, edges_hbm, zeros_hbm, out_hbm, sidx, didx, rows, acc_sh, gsem, ssem):
    s = lax.axis_index("s")
    wid = s

    pltpu.sync_copy(zeros_hbm, acc_sh.at[pl.ds(s * ROWS_PER_TILE, ROWS_PER_TILE)])
    pltpu.sync_copy(edges_hbm.at[0, wid, pl.ds(0, IB)], sidx)
    pltpu.sync_copy(edges_hbm.at[1, wid, pl.ds(0, IB)], didx)
    pltpu.async_copy(x_hbm.at[sidx.at[0]], rows[0], gsem[0])

    plsc.subcore_barrier()

    for kb in range(NIB1):
        if kb > 0:
            pltpu.sync_copy(edges_hbm.at[0, wid, pl.ds(kb * IB, IB)], sidx)
            pltpu.sync_copy(edges_hbm.at[1, wid, pl.ds(kb * IB, IB)], didx)
            pltpu.async_copy(x_hbm.at[sidx.at[0]], rows[0], gsem[0])

        def pair(t, _):
            for b in range(NB):
                jj = t * NB + b
                bp = 1 - b

                @pl.when(jj >= 1)
                def _recycle():
                    pltpu.make_async_copy(
                        rows[bp], acc_sh.at[didx.at[jj - 1]], ssem[bp]
                    ).wait()

                @pl.when(jj + 1 < IB)
                def _prefetch():
                    pltpu.async_copy(x_hbm.at[sidx.at[jj + 1]], rows[bp], gsem[bp])

                pltpu.make_async_copy(x_hbm.at[sidx.at[jj]], rows[b], gsem[b]).wait()
                pltpu.async_copy(rows[b], acc_sh.at[didx.at[jj]], ssem[b], add=True)
            return _

        lax.fori_loop(0, IB // NB, pair, None)

        pltpu.make_async_copy(rows[1], acc_sh.at[didx.at[IB - 1]], ssem[1]).wait()

    plsc.subcore_barrier()

    pltpu.sync_copy(
        acc_sh.at[pl.ds(s * ROWS_PER_TILE, ROWS_PER_TILE)],
        out_hbm.at[pl.ds(s * ROWS_PER_TILE, ROWS_PER_TILE)],
    )


_RB = 2048


def _tc_prep_body(deg_ref, feat_ref, z0_ref, norm_ref, norm2_ref):
    d = jnp.sum(deg_ref[...], axis=0)
    d = jnp.maximum(d, 1.0)
    nrm = lax.rsqrt(d)[:, None]
    norm_ref[...] = jnp.broadcast_to(nrm, (_RB, D))
    norm2_ref[...] = jnp.broadcast_to((1.0 / d)[:, None], (_RB, D))
    z0_ref[...] = feat_ref[...] * nrm


_tc_prep = pl.pallas_call(
    _tc_prep_body,
    grid=(NP // _RB,),
    in_specs=[
        pl.BlockSpec((NW, _RB), lambda i: (0, i)),
        pl.BlockSpec((_RB, D), lambda i: (i, 0)),
    ],
    out_specs=[
        pl.BlockSpec((_RB, D), lambda i: (i, 0)),
        pl.BlockSpec((_RB, D), lambda i: (i, 0)),
        pl.BlockSpec((_RB, D), lambda i: (i, 0)),
    ],
    out_shape=[
        jax.ShapeDtypeStruct((NP, D), jnp.float32),
        jax.ShapeDtypeStruct((NP, D), jnp.float32),
        jax.ShapeDtypeStruct((NP, D), jnp.float32),
    ],
)


def _tc_scale_body(acc_ref, norm2_ref, z1_ref):
    z1_ref[...] = acc_ref[...] * norm2_ref[...]


_tc_scale = pl.pallas_call(
    _tc_scale_body,
    grid=(NP // _RB,),
    in_specs=[
        pl.BlockSpec((_RB, D), lambda i: (i, 0)),
        pl.BlockSpec((_RB, D), lambda i: (i, 0)),
    ],
    out_specs=pl.BlockSpec((_RB, D), lambda i: (i, 0)),
    out_shape=jax.ShapeDtypeStruct((NP, D), jnp.float32),
)


def _tc_final_body(acc_ref, norm_ref, w_ref, b_ref, out_ref):
    h = acc_ref[...] * norm_ref[...]
    out_ref[...] = (
        jnp.dot(h, w_ref[...], preferred_element_type=jnp.float32) + b_ref[...]
    )


_tc_final = pl.pallas_call(
    _tc_final_body,
    grid=(NP // _RB,),
    in_specs=[
        pl.BlockSpec((_RB, D), lambda i: (i, 0)),
        pl.BlockSpec((_RB, D), lambda i: (i, 0)),
        pl.BlockSpec((D, D), lambda i: (0, 0)),
        pl.BlockSpec((1, D), lambda i: (0, 0)),
    ],
    out_specs=pl.BlockSpec((_RB, D), lambda i: (i, 0)),
    out_shape=jax.ShapeDtypeStruct((NP, D), jnp.float32),
)


def kernel(feat, edge_index, weight, bias):
    feat_p = jnp.pad(feat, ((0, NP - N), (0, 0)))
    edges_p = jnp.pad(edge_index, ((0, 0), (0, EPAD - E)), constant_values=NP - 1)
    edges_r = edges_p.reshape(2, NW, NCH, CH)
    edges_r1 = edges_p.reshape(2, NS, NCH1, CH)

    zeros_rows = jnp.zeros((ROWS_PER_TILE, D), jnp.float32)

    deg_parts = _sc_degree(edges_r)
    z0, norm_m, norm2_m = _tc_prep(deg_parts, feat_p)
    acc_a = _sc_hop(z0, edges_r1, zeros_rows)
    z1 = _tc_scale(acc_a, norm2_m)
    acc_b = _sc_hop(z1, edges_r1, zeros_rows)
    out = _tc_final(acc_b, norm_m, weight, bias.reshape(1, D))
    return out[:N]


# speedup vs baseline: 1.0785x; 1.0608x over previous
import functools

import jax
import jax.numpy as jnp
from jax import lax
from jax.experimental import pallas as pl
from jax.experimental.pallas import tpu as pltpu
from jax.experimental.pallas import tpu_sc as plsc

N = 10000
E = 320000
D = 128

_INFO = plsc.get_sparse_core_info()
NC = _INFO.num_cores
NS = _INFO.num_subcores
NW = NC * NS

NP = 10240
ROWS_PER_TILE = NP // NS
CH = 64
NCH = 160
NB = 4
NG = NB // 2
NSC = NB - NG
IB = 40
NIB = NCH // IB
EPT = NCH * CH
EPAD = EPT * NW

_MESH = plsc.VectorSubcoreMesh(core_axis_name="c", subcore_axis_name="s")
_SC_PARAMS = pltpu.CompilerParams(needs_layout_passes=False)


@functools.partial(
    pl.kernel,
    out_type=jax.ShapeDtypeStruct((NW, NP), jnp.float32),
    mesh=_MESH,
    scratch_types=[
        pltpu.VMEM((NCH, CH), jnp.int32),
        pltpu.VMEM((NP,), jnp.float32),
    ],
    compiler_params=_SC_PARAMS,
)
def _sc_degree(edges_hbm, out_hbm, dst_v, hist_v):
    c = lax.axis_index("c")
    s = lax.axis_index("s")
    wid = s * NC + c

    def zero(i, _):
        hist_v[pl.ds(i * 16, 16)] = jnp.zeros((16,), jnp.float32)
        return _

    lax.fori_loop(0, NP // 16, zero, None)

    pltpu.sync_copy(edges_hbm.at[1, wid], dst_v)

    ones = jnp.ones((16,), jnp.float32)

    def row(r, _):
        def col(k, __):
            idx = dst_v[r, pl.ds(k * 16, 16)]
            plsc.addupdate_scatter(hist_v, [idx], ones)
            return __

        return lax.fori_loop(0, CH // 16, col, _)

    lax.fori_loop(0, NCH, row, None)

    pltpu.sync_copy(hist_v, out_hbm.at[wid])


@functools.partial(
    pl.kernel,
    out_type=jax.ShapeDtypeStruct((NC, NP, D), jnp.float32),
    mesh=_MESH,
    scratch_types=[
        pltpu.VMEM((IB, CH), jnp.int32),
        pltpu.VMEM((IB, CH), jnp.int32),
        [pltpu.VMEM((CH, D), jnp.float32)] * NB,
        pltpu.VMEM_SHARED((NP, D), jnp.float32),
        [pltpu.SemaphoreType.DMA] * NB,
        [pltpu.SemaphoreType.DMA] * NB,
    ],
    compiler_params=_SC_PARAMS,
)
def _sc_hop(x_hbm, edges_hbm, zeros_hbm, out_hbm, sidx, didx, rows, acc_sh, gsem, ssem):
    c = lax.axis_index("c")
    s = lax.axis_index("s")
    wid = s * NC + c

    pltpu.sync_copy(zeros_hbm, acc_sh.at[pl.ds(s * ROWS_PER_TILE, ROWS_PER_TILE)])
    pltpu.sync_copy(edges_hbm.at[0, wid, pl.ds(0, IB)], sidx)
    pltpu.sync_copy(edges_hbm.at[1, wid, pl.ds(0, IB)], didx)
    for b in range(NG):
        pltpu.async_copy(x_hbm.at[sidx.at[b]], rows[b], gsem[b])

    plsc.subcore_barrier()

    for kb in range(NIB):
        if kb > 0:
            pltpu.sync_copy(edges_hbm.at[0, wid, pl.ds(kb * IB, IB)], sidx)
            pltpu.sync_copy(edges_hbm.at[1, wid, pl.ds(kb * IB, IB)], didx)
            for b in range(NG):
                pltpu.async_copy(x_hbm.at[sidx.at[b]], rows[b], gsem[b])

        def group(t, _):
            for b in range(NB):
                jj = t * NB + b
                br = (b + NG) % NB

                @pl.when(jj >= NSC)
                def _recycle():
                    pltpu.make_async_copy(
                        rows[br], acc_sh.at[didx.at[jj - NSC]], ssem[br]
                    ).wait()

                @pl.when(jj + NG < IB)
                def _prefetch():
                    pltpu.async_copy(x_hbm.at[sidx.at[jj + NG]], rows[br], gsem[br])

                pltpu.make_async_copy(x_hbm.at[sidx.at[jj]], rows[b], gsem[b]).wait()
                pltpu.async_copy(rows[b], acc_sh.at[didx.at[jj]], ssem[b], add=True)
            return _

        lax.fori_loop(0, IB // NB, group, None)

        for k in range(NSC):
            jj = IB - NSC + k
            pltpu.make_async_copy(
                rows[jj % NB], acc_sh.at[didx.at[jj]], ssem[jj % NB]
            ).wait()

    plsc.subcore_barrier()

    pltpu.sync_copy(
        acc_sh.at[pl.ds(s * ROWS_PER_TILE, ROWS_PER_TILE)],
        out_hbm.at[c, pl.ds(s * ROWS_PER_TILE, ROWS_PER_TILE)],
    )


_RB = 2048


def _tc_prep_body(deg_ref, feat_ref, z0_ref, norm_ref, norm2_ref):
    d = jnp.sum(deg_ref[...], axis=0)
    d = jnp.maximum(d, 1.0)
    nrm = lax.rsqrt(d)[:, None]
    norm_ref[...] = jnp.broadcast_to(nrm, (_RB, D))
    norm2_ref[...] = jnp.broadcast_to((1.0 / d)[:, None], (_RB, D))
    z0_ref[...] = feat_ref[...] * nrm


_tc_prep = pl.pallas_call(
    _tc_prep_body,
    grid=(NP // _RB,),
    in_specs=[
        pl.BlockSpec((NW, _RB), lambda i: (0, i)),
        pl.BlockSpec((_RB, D), lambda i: (i, 0)),
    ],
    out_specs=[
        pl.BlockSpec((_RB, D), lambda i: (i, 0)),
        pl.BlockSpec((_RB, D), lambda i: (i, 0)),
        pl.BlockSpec((_RB, D), lambda i: (i, 0)),
    ],
    out_shape=[
        jax.ShapeDtypeStruct((NP, D), jnp.float32),
        jax.ShapeDtypeStruct((NP, D), jnp.float32),
        jax.ShapeDtypeStruct((NP, D), jnp.float32),
    ],
)


def _tc_scale_body(acc_ref, norm2_ref, z1_ref):
    z1_ref[...] = (acc_ref[0] + acc_ref[1]) * norm2_ref[...]


_tc_scale = pl.pallas_call(
    _tc_scale_body,
    grid=(NP // _RB,),
    in_specs=[
        pl.BlockSpec((NC, _RB, D), lambda i: (0, i, 0)),
        pl.BlockSpec((_RB, D), lambda i: (i, 0)),
    ],
    out_specs=pl.BlockSpec((_RB, D), lambda i: (i, 0)),
    out_shape=jax.ShapeDtypeStruct((NP, D), jnp.float32),
)


def _tc_final_body(acc_ref, norm_ref, w_ref, b_ref, out_ref):
    h = (acc_ref[0] + acc_ref[1]) * norm_ref[...]
    out_ref[...] = (
        jnp.dot(h, w_ref[...], preferred_element_type=jnp.float32) + b_ref[...]
    )


_tc_final = pl.pallas_call(
    _tc_final_body,
    grid=(NP // _RB,),
    in_specs=[
        pl.BlockSpec((NC, _RB, D), lambda i: (0, i, 0)),
        pl.BlockSpec((_RB, D), lambda i: (i, 0)),
        pl.BlockSpec((D, D), lambda i: (0, 0)),
        pl.BlockSpec((1, D), lambda i: (0, 0)),
    ],
    out_specs=pl.BlockSpec((_RB, D), lambda i: (i, 0)),
    out_shape=jax.ShapeDtypeStruct((NP, D), jnp.float32),
)


def kernel(feat, edge_index, weight, bias):
    feat_p = jnp.pad(feat, ((0, NP - N), (0, 0)))
    edges_p = jnp.pad(edge_index, ((0, 0), (0, EPAD - E)), constant_values=NP - 1)
    edges_r = edges_p.reshape(2, NW, NCH, CH)

    zeros_rows = jnp.zeros((ROWS_PER_TILE, D), jnp.float32)

    deg_parts = _sc_degree(edges_r)
    z0, norm_m, norm2_m = _tc_prep(deg_parts, feat_p)
    acc_a = _sc_hop(z0, edges_r, zeros_rows)
    z1 = _tc_scale(acc_a, norm2_m)
    acc_b = _sc_hop(z1, edges_r, zeros_rows)
    out = _tc_final(acc_b, norm_m, weight, bias.reshape(1, D))
    return out[:N]


# speedup vs baseline: 3.6256x; 3.3616x over previous
import functools

import jax
import jax.numpy as jnp
from jax import lax
from jax.experimental import pallas as pl
from jax.experimental.pallas import tpu as pltpu
from jax.experimental.pallas import tpu_sc as plsc

N = 10000
E = 320000
D = 128

_INFO = plsc.get_sparse_core_info()
NC = _INFO.num_cores
NS = _INFO.num_subcores
NW = NC * NS

NP = 10240
ROWS_PER_TILE = NP // NS
CH = 64
NCH = 160
NB = 4
NG = NB // 2
NSC = NB - NG
IB = 40
NIB = NCH // IB
EPT = NCH * CH
EPAD = EPT * NW

_MESH = plsc.VectorSubcoreMesh(core_axis_name="c", subcore_axis_name="s")
_SC_PARAMS = pltpu.CompilerParams(needs_layout_passes=False)


@functools.partial(
    pl.kernel,
    out_type=jax.ShapeDtypeStruct((NW, NP), jnp.float32),
    mesh=_MESH,
    scratch_types=[
        pltpu.VMEM((NCH, CH), jnp.int32),
        pltpu.VMEM((NP,), jnp.float32),
    ],
    compiler_params=_SC_PARAMS,
)
def _sc_degree(edges_hbm, out_hbm, dst_v, hist_v):
    c = lax.axis_index("c")
    s = lax.axis_index("s")
    wid = s * NC + c

    def zero(i, _):
        hist_v[pl.ds(i * 16, 16)] = jnp.zeros((16,), jnp.float32)
        return _

    lax.fori_loop(0, NP // 16, zero, None)

    pltpu.sync_copy(edges_hbm.at[1, wid], dst_v)

    ones = jnp.ones((16,), jnp.float32)

    def row(r, _):
        def col(k, __):
            idx = dst_v[r, pl.ds(k * 16, 16)]
            plsc.addupdate_scatter(hist_v, [idx], ones)
            return __

        return lax.fori_loop(0, CH // 16, col, _)

    lax.fori_loop(0, NCH, row, None)

    pltpu.sync_copy(hist_v, out_hbm.at[wid])


@functools.partial(
    pl.kernel,
    out_type=jax.ShapeDtypeStruct((NC, NP, D), jnp.float32),
    mesh=_MESH,
    scratch_types=[
        pltpu.VMEM((IB, CH), jnp.int32),
        pltpu.VMEM((IB, CH), jnp.int32),
        [pltpu.VMEM((CH, D), jnp.float32)] * NB,
        pltpu.VMEM_SHARED((NP, D), jnp.float32),
        [pltpu.SemaphoreType.DMA] * NB,
        [pltpu.SemaphoreType.DMA] * NB,
    ],
    compiler_params=_SC_PARAMS,
)
def _sc_hop(x_hbm, edges_hbm, zeros_hbm, out_hbm, sidx, didx, rows, acc_sh, gsem, ssem):
    c = lax.axis_index("c")
    s = lax.axis_index("s")
    wid = s * NC + c

    pltpu.sync_copy(zeros_hbm, acc_sh.at[pl.ds(s * ROWS_PER_TILE, ROWS_PER_TILE)])
    pltpu.sync_copy(edges_hbm.at[0, wid, pl.ds(0, IB)], sidx)
    pltpu.sync_copy(edges_hbm.at[1, wid, pl.ds(0, IB)], didx)
    for b in range(NG):
        pltpu.async_copy(x_hbm.at[sidx.at[b]], rows[b], gsem[b])

    plsc.subcore_barrier()

    for kb in range(NIB):
        if kb > 0:
            pltpu.sync_copy(edges_hbm.at[0, wid, pl.ds(kb * IB, IB)], sidx)
            pltpu.sync_copy(edges_hbm.at[1, wid, pl.ds(kb * IB, IB)], didx)
            for b in range(NG):
                pltpu.async_copy(x_hbm.at[sidx.at[b]], rows[b], gsem[b])

        def group(t, _):
            for b in range(NB):
                jj = t * NB + b
                br = (b + NG) % NB

                @pl.when(jj >= NSC)
                def _recycle():
                    pltpu.make_async_copy(
                        rows[br], acc_sh.at[didx.at[jj - NSC]], ssem[br]
                    ).wait()

                @pl.when(jj + NG < IB)
                def _prefetch():
                    pltpu.async_copy(x_hbm.at[sidx.at[jj + NG]], rows[br], gsem[br])

                pltpu.make_async_copy(x_hbm.at[sidx.at[jj]], rows[b], gsem[b]).wait()
                pltpu.async_copy(rows[b], acc_sh.at[didx.at[jj]], ssem[b], add=True)
            return _

        lax.fori_loop(0, IB // NB, group, None)

        for k in range(NSC):
            jj = IB - NSC + k
            pltpu.make_async_copy(
                rows[jj % NB], acc_sh.at[didx.at[jj]], ssem[jj % NB]
            ).wait()

    plsc.subcore_barrier()

    pltpu.sync_copy(
        acc_sh.at[pl.ds(s * ROWS_PER_TILE, ROWS_PER_TILE)],
        out_hbm.at[c, pl.ds(s * ROWS_PER_TILE, ROWS_PER_TILE)],
    )


_RB = 2048


def _tc_prep_body(deg_ref, feat_ref, z0_ref, norm_ref, norm2_ref):
    d = jnp.sum(deg_ref[...], axis=0)
    d = jnp.maximum(d, 1.0)
    nrm = lax.rsqrt(d)[:, None]
    norm_ref[...] = jnp.broadcast_to(nrm, (_RB, D))
    norm2_ref[...] = jnp.broadcast_to((1.0 / d)[:, None], (_RB, D))
    z0_ref[...] = feat_ref[...] * nrm


_tc_prep = pl.pallas_call(
    _tc_prep_body,
    grid=(NP // _RB,),
    in_specs=[
        pl.BlockSpec((NW, _RB), lambda i: (0, i)),
        pl.BlockSpec((_RB, D), lambda i: (i, 0)),
    ],
    out_specs=[
        pl.BlockSpec((_RB, D), lambda i: (i, 0)),
        pl.BlockSpec((_RB, D), lambda i: (i, 0)),
        pl.BlockSpec((_RB, D), lambda i: (i, 0)),
    ],
    out_shape=[
        jax.ShapeDtypeStruct((NP, D), jnp.float32),
        jax.ShapeDtypeStruct((NP, D), jnp.float32),
        jax.ShapeDtypeStruct((NP, D), jnp.float32),
    ],
)


def _tc_scale_body(acc_ref, norm2_ref, z1_ref):
    z1_ref[...] = (acc_ref[0] + acc_ref[1]) * norm2_ref[...]


_tc_scale = pl.pallas_call(
    _tc_scale_body,
    grid=(NP // _RB,),
    in_specs=[
        pl.BlockSpec((NC, _RB, D), lambda i: (0, i, 0)),
        pl.BlockSpec((_RB, D), lambda i: (i, 0)),
    ],
    out_specs=pl.BlockSpec((_RB, D), lambda i: (i, 0)),
    out_shape=jax.ShapeDtypeStruct((NP, D), jnp.float32),
)


def _tc_final_body(acc_ref, norm_ref, w_ref, b_ref, out_ref):
    h = (acc_ref[0] + acc_ref[1]) * norm_ref[...]
    out_ref[...] = (
        jnp.dot(h, w_ref[...], preferred_element_type=jnp.float32) + b_ref[...]
    )


_tc_final = pl.pallas_call(
    _tc_final_body,
    grid=(NP // _RB,),
    in_specs=[
        pl.BlockSpec((NC, _RB, D), lambda i: (0, i, 0)),
        pl.BlockSpec((_RB, D), lambda i: (i, 0)),
        pl.BlockSpec((D, D), lambda i: (0, 0)),
        pl.BlockSpec((1, D), lambda i: (0, 0)),
    ],
    out_specs=pl.BlockSpec((_RB, D), lambda i: (i, 0)),
    out_shape=jax.ShapeDtypeStruct((NP, D), jnp.float32),
)


def kernel(feat, edge_index, weight, bias):
    feat_p = jnp.pad(feat, ((0, NP - N), (0, 0)))
    pad_idx = (jnp.arange(EPAD - E, dtype=jnp.int32) % (NP - N)) + N
    edges_p = jnp.concatenate(
        [edge_index, jnp.stack([pad_idx, pad_idx])], axis=1
    )
    edges_r = edges_p.reshape(2, NW, NCH, CH)

    zeros_rows = jnp.zeros((ROWS_PER_TILE, D), jnp.float32)

    deg_parts = _sc_degree(edges_r)
    z0, norm_m, norm2_m = _tc_prep(deg_parts, feat_p)
    acc_a = _sc_hop(z0, edges_r, zeros_rows)
    z1 = _tc_scale(acc_a, norm2_m)
    acc_b = _sc_hop(z1, edges_r, zeros_rows)
    out = _tc_final(acc_b, norm_m, weight, bias.reshape(1, D))
    return out[:N]


# speedup vs baseline: 3.7757x; 1.0414x over previous
import functools

import jax
import jax.numpy as jnp
from jax import lax
from jax.experimental import pallas as pl
from jax.experimental.pallas import tpu as pltpu
from jax.experimental.pallas import tpu_sc as plsc

N = 10000
E = 320000
D = 128

_INFO = plsc.get_sparse_core_info()
NC = _INFO.num_cores
NS = _INFO.num_subcores
NW = NC * NS

NP = 10240
ROWS_PER_TILE = NP // NS
CH = 128
NCH = 80
NB = 2
NG = NB // 2
NSC = NB - NG
IB = 40
NIB = NCH // IB
EPT = NCH * CH
EPAD = EPT * NW

_MESH = plsc.VectorSubcoreMesh(core_axis_name="c", subcore_axis_name="s")
_SC_PARAMS = pltpu.CompilerParams(needs_layout_passes=False)


@functools.partial(
    pl.kernel,
    out_type=jax.ShapeDtypeStruct((NW, NP), jnp.float32),
    mesh=_MESH,
    scratch_types=[
        pltpu.VMEM((NCH, CH), jnp.int32),
        pltpu.VMEM((NP,), jnp.float32),
    ],
    compiler_params=_SC_PARAMS,
)
def _sc_degree(edges_hbm, out_hbm, dst_v, hist_v):
    c = lax.axis_index("c")
    s = lax.axis_index("s")
    wid = s * NC + c

    def zero(i, _):
        hist_v[pl.ds(i * 16, 16)] = jnp.zeros((16,), jnp.float32)
        return _

    lax.fori_loop(0, NP // 16, zero, None)

    pltpu.sync_copy(edges_hbm.at[1, wid], dst_v)

    ones = jnp.ones((16,), jnp.float32)

    def row(r, _):
        def col(k, __):
            idx = dst_v[r, pl.ds(k * 16, 16)]
            plsc.addupdate_scatter(hist_v, [idx], ones)
            return __

        return lax.fori_loop(0, CH // 16, col, _)

    lax.fori_loop(0, NCH, row, None)

    pltpu.sync_copy(hist_v, out_hbm.at[wid])


@functools.partial(
    pl.kernel,
    out_type=jax.ShapeDtypeStruct((NC, NP, D), jnp.float32),
    mesh=_MESH,
    scratch_types=[
        pltpu.VMEM((IB, CH), jnp.int32),
        pltpu.VMEM((IB, CH), jnp.int32),
        [pltpu.VMEM((CH, D), jnp.float32)] * NB,
        pltpu.VMEM_SHARED((NP, D), jnp.float32),
        [pltpu.SemaphoreType.DMA] * NB,
        [pltpu.SemaphoreType.DMA] * NB,
    ],
    compiler_params=_SC_PARAMS,
)
def _sc_hop(x_hbm, edges_hbm, zeros_hbm, out_hbm, sidx, didx, rows, acc_sh, gsem, ssem):
    c = lax.axis_index("c")
    s = lax.axis_index("s")
    wid = s * NC + c

    pltpu.sync_copy(zeros_hbm, acc_sh.at[pl.ds(s * ROWS_PER_TILE, ROWS_PER_TILE)])
    pltpu.sync_copy(edges_hbm.at[0, wid, pl.ds(0, IB)], sidx)
    pltpu.sync_copy(edges_hbm.at[1, wid, pl.ds(0, IB)], didx)
    for b in range(NG):
        pltpu.async_copy(x_hbm.at[sidx.at[b]], rows[b], gsem[b])

    plsc.subcore_barrier()

    for kb in range(NIB):
        if kb > 0:
            pltpu.sync_copy(edges_hbm.at[0, wid, pl.ds(kb * IB, IB)], sidx)
            pltpu.sync_copy(edges_hbm.at[1, wid, pl.ds(kb * IB, IB)], didx)
            for b in range(NG):
                pltpu.async_copy(x_hbm.at[sidx.at[b]], rows[b], gsem[b])

        def group(t, _):
            for b in range(NB):
                jj = t * NB + b
                br = (b + NG) % NB

                @pl.when(jj >= NSC)
                def _recycle():
                    pltpu.make_async_copy(
                        rows[br], acc_sh.at[didx.at[jj - NSC]], ssem[br]
                    ).wait()

                @pl.when(jj + NG < IB)
                def _prefetch():
                    pltpu.async_copy(x_hbm.at[sidx.at[jj + NG]], rows[br], gsem[br])

                pltpu.make_async_copy(x_hbm.at[sidx.at[jj]], rows[b], gsem[b]).wait()
                pltpu.async_copy(rows[b], acc_sh.at[didx.at[jj]], ssem[b], add=True)
            return _

        lax.fori_loop(0, IB // NB, group, None)

        for k in range(NSC):
            jj = IB - NSC + k
            pltpu.make_async_copy(
                rows[jj % NB], acc_sh.at[didx.at[jj]], ssem[jj % NB]
            ).wait()

    plsc.subcore_barrier()

    pltpu.sync_copy(
        acc_sh.at[pl.ds(s * ROWS_PER_TILE, ROWS_PER_TILE)],
        out_hbm.at[c, pl.ds(s * ROWS_PER_TILE, ROWS_PER_TILE)],
    )


_RB = 2048


def _tc_prep_body(deg_ref, feat_ref, z0_ref, norm_ref, norm2_ref):
    d = jnp.sum(deg_ref[...], axis=0)
    d = jnp.maximum(d, 1.0)
    nrm = lax.rsqrt(d)[:, None]
    norm_ref[...] = jnp.broadcast_to(nrm, (_RB, D))
    norm2_ref[...] = jnp.broadcast_to((1.0 / d)[:, None], (_RB, D))
    z0_ref[...] = feat_ref[...] * nrm


_tc_prep = pl.pallas_call(
    _tc_prep_body,
    grid=(NP // _RB,),
    in_specs=[
        pl.BlockSpec((NW, _RB), lambda i: (0, i)),
        pl.BlockSpec((_RB, D), lambda i: (i, 0)),
    ],
    out_specs=[
        pl.BlockSpec((_RB, D), lambda i: (i, 0)),
        pl.BlockSpec((_RB, D), lambda i: (i, 0)),
        pl.BlockSpec((_RB, D), lambda i: (i, 0)),
    ],
    out_shape=[
        jax.ShapeDtypeStruct((NP, D), jnp.float32),
        jax.ShapeDtypeStruct((NP, D), jnp.float32),
        jax.ShapeDtypeStruct((NP, D), jnp.float32),
    ],
)


def _tc_scale_body(acc_ref, norm2_ref, z1_ref):
    z1_ref[...] = (acc_ref[0] + acc_ref[1]) * norm2_ref[...]


_tc_scale = pl.pallas_call(
    _tc_scale_body,
    grid=(NP // _RB,),
    in_specs=[
        pl.BlockSpec((NC, _RB, D), lambda i: (0, i, 0)),
        pl.BlockSpec((_RB, D), lambda i: (i, 0)),
    ],
    out_specs=pl.BlockSpec((_RB, D), lambda i: (i, 0)),
    out_shape=jax.ShapeDtypeStruct((NP, D), jnp.float32),
)


def _tc_final_body(acc_ref, norm_ref, w_ref, b_ref, out_ref):
    h = (acc_ref[0] + acc_ref[1]) * norm_ref[...]
    out_ref[...] = (
        jnp.dot(h, w_ref[...], preferred_element_type=jnp.float32) + b_ref[...]
    )


_tc_final = pl.pallas_call(
    _tc_final_body,
    grid=(NP // _RB,),
    in_specs=[
        pl.BlockSpec((NC, _RB, D), lambda i: (0, i, 0)),
        pl.BlockSpec((_RB, D), lambda i: (i, 0)),
        pl.BlockSpec((D, D), lambda i: (0, 0)),
        pl.BlockSpec((1, D), lambda i: (0, 0)),
    ],
    out_specs=pl.BlockSpec((_RB, D), lambda i: (i, 0)),
    out_shape=jax.ShapeDtypeStruct((NP, D), jnp.float32),
)


def kernel(feat, edge_index, weight, bias):
    feat_p = jnp.pad(feat, ((0, NP - N), (0, 0)))
    pad_idx = (jnp.arange(EPAD - E, dtype=jnp.int32) % (NP - N)) + N
    edges_p = jnp.concatenate(
        [edge_index, jnp.stack([pad_idx, pad_idx])], axis=1
    )
    edges_r = edges_p.reshape(2, NW, NCH, CH)

    zeros_rows = jnp.zeros((ROWS_PER_TILE, D), jnp.float32)

    deg_parts = _sc_degree(edges_r)
    z0, norm_m, norm2_m = _tc_prep(deg_parts, feat_p)
    acc_a = _sc_hop(z0, edges_r, zeros_rows)
    z1 = _tc_scale(acc_a, norm2_m)
    acc_b = _sc_hop(z1, edges_r, zeros_rows)
    out = _tc_final(acc_b, norm_m, weight, bias.reshape(1, D))
    return out[:N]


# speedup vs baseline: 3.7882x; 1.0033x over previous
import functools

import jax
import jax.numpy as jnp
from jax import lax
from jax.experimental import pallas as pl
from jax.experimental.pallas import tpu as pltpu
from jax.experimental.pallas import tpu_sc as plsc

N = 10000
E = 320000
D = 128

_INFO = plsc.get_sparse_core_info()
NC = _INFO.num_cores
NS = _INFO.num_subcores
NW = NC * NS

NP = 10240
ROWS_PER_TILE = NP // NS
CH = 128
NCH = 80
NB = 2
NG = NB // 2
NSC = NB - NG
IB = 40
NIB = NCH // IB
EPT = NCH * CH
EPAD = EPT * NW

_MESH = plsc.VectorSubcoreMesh(core_axis_name="c", subcore_axis_name="s")
_SC_PARAMS = pltpu.CompilerParams(needs_layout_passes=False)


@functools.partial(
    pl.kernel,
    out_type=jax.ShapeDtypeStruct((NW, NP), jnp.float32),
    mesh=_MESH,
    scratch_types=[
        pltpu.VMEM((NCH, CH), jnp.int32),
        pltpu.VMEM((NP,), jnp.float32),
    ],
    compiler_params=_SC_PARAMS,
)
def _sc_degree(edges_hbm, out_hbm, dst_v, hist_v):
    c = lax.axis_index("c")
    s = lax.axis_index("s")
    wid = s * NC + c

    def zero(i, _):
        hist_v[pl.ds(i * 16, 16)] = jnp.zeros((16,), jnp.float32)
        return _

    lax.fori_loop(0, NP // 16, zero, None)

    pltpu.sync_copy(edges_hbm.at[1, wid], dst_v)

    ones = jnp.ones((16,), jnp.float32)

    def row(r, _):
        def col(k, __):
            idx = dst_v[r, pl.ds(k * 16, 16)]
            plsc.addupdate_scatter(hist_v, [idx], ones)
            return __

        return lax.fori_loop(0, CH // 16, col, _)

    lax.fori_loop(0, NCH, row, None)

    pltpu.sync_copy(hist_v, out_hbm.at[wid])


@functools.partial(
    pl.kernel,
    out_type=jax.ShapeDtypeStruct((NC, NP, D), jnp.float32),
    mesh=_MESH,
    scratch_types=[
        pltpu.VMEM((IB, CH), jnp.int32),
        pltpu.VMEM((IB, CH), jnp.int32),
        [pltpu.VMEM((CH, D), jnp.float32)] * NB,
        pltpu.VMEM_SHARED((NP, D), jnp.float32),
        [pltpu.SemaphoreType.DMA] * NB,
        [pltpu.SemaphoreType.DMA] * NB,
    ],
    compiler_params=_SC_PARAMS,
)
def _sc_hop(x_hbm, edges_hbm, zeros_hbm, out_hbm, sidx, didx, rows, acc_sh, gsem, ssem):
    c = lax.axis_index("c")
    s = lax.axis_index("s")
    wid = s * NC + c

    pltpu.sync_copy(zeros_hbm, acc_sh.at[pl.ds(s * ROWS_PER_TILE, ROWS_PER_TILE)])
    pltpu.sync_copy(edges_hbm.at[0, wid, pl.ds(0, IB)], sidx)
    pltpu.sync_copy(edges_hbm.at[1, wid, pl.ds(0, IB)], didx)
    for b in range(NG):
        pltpu.async_copy(x_hbm.at[sidx.at[b]], rows[b], gsem[b])

    plsc.subcore_barrier()

    for kb in range(NIB):
        if kb > 0:
            pltpu.sync_copy(edges_hbm.at[0, wid, pl.ds(kb * IB, IB)], sidx)
            pltpu.sync_copy(edges_hbm.at[1, wid, pl.ds(kb * IB, IB)], didx)
            for b in range(NG):
                pltpu.async_copy(x_hbm.at[sidx.at[b]], rows[b], gsem[b])

        def group(t, _):
            for b in range(NB):
                jj = t * NB + b
                br = (b + NG) % NB

                @pl.when(jj >= NSC)
                def _recycle():
                    pltpu.make_async_copy(
                        rows[br], acc_sh.at[didx.at[jj - NSC]], ssem[br]
                    ).wait()

                @pl.when(jj + NG < IB)
                def _prefetch():
                    pltpu.async_copy(x_hbm.at[sidx.at[jj + NG]], rows[br], gsem[br])

                pltpu.make_async_copy(x_hbm.at[sidx.at[jj]], rows[b], gsem[b]).wait()
                pltpu.async_copy(rows[b], acc_sh.at[didx.at[jj]], ssem[b], add=True)
            return _

        lax.fori_loop(0, IB // NB, group, None)

        for k in range(NSC):
            jj = IB - NSC + k
            pltpu.make_async_copy(
                rows[jj % NB], acc_sh.at[didx.at[jj]], ssem[jj % NB]
            ).wait()

    plsc.subcore_barrier()

    pltpu.sync_copy(
        acc_sh.at[pl.ds(s * ROWS_PER_TILE, ROWS_PER_TILE)],
        out_hbm.at[c, pl.ds(s * ROWS_PER_TILE, ROWS_PER_TILE)],
    )


_RB = 2048


def _deg_of(deg_ref):
    return jnp.maximum(jnp.sum(deg_ref[...], axis=0), 1.0)


def _tc_prep_body(deg_ref, feat_ref, z0_ref):
    nrm = lax.rsqrt(_deg_of(deg_ref))[:, None]
    z0_ref[...] = feat_ref[...] * nrm


_tc_prep = pl.pallas_call(
    _tc_prep_body,
    grid=(NP // _RB,),
    in_specs=[
        pl.BlockSpec((NW, _RB), lambda i: (0, i)),
        pl.BlockSpec((_RB, D), lambda i: (i, 0)),
    ],
    out_specs=pl.BlockSpec((_RB, D), lambda i: (i, 0)),
    out_shape=jax.ShapeDtypeStruct((NP, D), jnp.float32),
)


def _tc_scale_body(acc_ref, deg_ref, z1_ref):
    inv = (1.0 / _deg_of(deg_ref))[:, None]
    z1_ref[...] = (acc_ref[0] + acc_ref[1]) * inv


_tc_scale = pl.pallas_call(
    _tc_scale_body,
    grid=(NP // _RB,),
    in_specs=[
        pl.BlockSpec((NC, _RB, D), lambda i: (0, i, 0)),
        pl.BlockSpec((NW, _RB), lambda i: (0, i)),
    ],
    out_specs=pl.BlockSpec((_RB, D), lambda i: (i, 0)),
    out_shape=jax.ShapeDtypeStruct((NP, D), jnp.float32),
)


def _tc_final_body(acc_ref, deg_ref, w_ref, b_ref, out_ref):
    nrm = lax.rsqrt(_deg_of(deg_ref))[:, None]
    h = (acc_ref[0] + acc_ref[1]) * nrm
    out_ref[...] = (
        jnp.dot(h, w_ref[...], preferred_element_type=jnp.float32) + b_ref[...]
    )


_tc_final = pl.pallas_call(
    _tc_final_body,
    grid=(NP // _RB,),
    in_specs=[
        pl.BlockSpec((NC, _RB, D), lambda i: (0, i, 0)),
        pl.BlockSpec((NW, _RB), lambda i: (0, i)),
        pl.BlockSpec((D, D), lambda i: (0, 0)),
        pl.BlockSpec((1, D), lambda i: (0, 0)),
    ],
    out_specs=pl.BlockSpec((_RB, D), lambda i: (i, 0)),
    out_shape=jax.ShapeDtypeStruct((NP, D), jnp.float32),
)


def kernel(feat, edge_index, weight, bias):
    feat_p = jnp.pad(feat, ((0, NP - N), (0, 0)))
    pad_idx = (jnp.arange(EPAD - E, dtype=jnp.int32) % (NP - N)) + N
    edges_p = jnp.concatenate([edge_index, jnp.stack([pad_idx, pad_idx])], axis=1)
    edges_r = edges_p.reshape(2, NW, NCH, CH)

    zeros_rows = jnp.zeros((ROWS_PER_TILE, D), jnp.float32)

    deg_parts = _sc_degree(edges_r)
    z0 = _tc_prep(deg_parts, feat_p)
    acc_a = _sc_hop(z0, edges_r, zeros_rows)
    z1 = _tc_scale(acc_a, deg_parts)
    acc_b = _sc_hop(z1, edges_r, zeros_rows)
    out = _tc_final(acc_b, deg_parts, weight, bias.reshape(1, D))
    return out[:N]


# speedup vs baseline: 3.8762x; 1.0232x over previous
import functools

import jax
import jax.numpy as jnp
from jax import lax
from jax.experimental import pallas as pl
from jax.experimental.pallas import tpu as pltpu
from jax.experimental.pallas import tpu_sc as plsc

N = 10000
E = 320000
D = 128

_INFO = plsc.get_sparse_core_info()
NC = _INFO.num_cores
NS = _INFO.num_subcores
NW = NC * NS

NP = 10240
ROWS_PER_TILE = NP // NS
CH = 128
NCH = 80
NB = 2
NG = NB // 2
NSC = NB - NG
IB = 40
NIB = NCH // IB
EPT = NCH * CH
EPAD = EPT * NW

_MESH = plsc.VectorSubcoreMesh(core_axis_name="c", subcore_axis_name="s")
_SC_PARAMS = pltpu.CompilerParams(needs_layout_passes=False)


@functools.partial(
    pl.kernel,
    out_type=jax.ShapeDtypeStruct((NW, NP), jnp.float32),
    mesh=_MESH,
    scratch_types=[
        pltpu.VMEM((NCH, CH), jnp.int32),
        pltpu.VMEM((NP,), jnp.float32),
    ],
    compiler_params=_SC_PARAMS,
)
def _sc_degree(edges_hbm, out_hbm, dst_v, hist_v):
    c = lax.axis_index("c")
    s = lax.axis_index("s")
    wid = s * NC + c

    def zero(i, _):
        hist_v[pl.ds(i * 16, 16)] = jnp.zeros((16,), jnp.float32)
        return _

    lax.fori_loop(0, NP // 16, zero, None)

    pltpu.sync_copy(edges_hbm.at[1, wid], dst_v)

    ones = jnp.ones((16,), jnp.float32)

    def row(r, _):
        def col(k, __):
            idx = dst_v[r, pl.ds(k * 16, 16)]
            plsc.addupdate_scatter(hist_v, [idx], ones)
            return __

        return lax.fori_loop(0, CH // 16, col, _)

    lax.fori_loop(0, NCH, row, None)

    pltpu.sync_copy(hist_v, out_hbm.at[wid])


@functools.partial(
    pl.kernel,
    out_type=jax.ShapeDtypeStruct((NC, NP, D), jnp.float32),
    mesh=_MESH,
    scratch_types=[
        pltpu.VMEM((IB, CH), jnp.int32),
        pltpu.VMEM((IB, CH), jnp.int32),
        [pltpu.VMEM((CH, D), jnp.float32)] * NB,
        pltpu.VMEM_SHARED((NP, D), jnp.float32),
        [pltpu.SemaphoreType.DMA] * NB,
        [pltpu.SemaphoreType.DMA] * NB,
    ],
    compiler_params=_SC_PARAMS,
)
def _sc_hop(x_hbm, edges_hbm, out_hbm, sidx, didx, rows, acc_sh, gsem, ssem):
    c = lax.axis_index("c")
    s = lax.axis_index("s")
    wid = s * NC + c

    def zrow(r, _):
        def zcol(k, __):
            rows[0][r, pl.ds(k * 16, 16)] = jnp.zeros((16,), jnp.float32)
            return __

        return lax.fori_loop(0, D // 16, zcol, _)

    lax.fori_loop(0, CH, zrow, None)

    def zcopy(i, _):
        pltpu.sync_copy(rows[0], acc_sh.at[pl.ds(s * ROWS_PER_TILE + i * CH, CH)])
        return _

    lax.fori_loop(0, ROWS_PER_TILE // CH, zcopy, None)

    pltpu.sync_copy(edges_hbm.at[0, wid, pl.ds(0, IB)], sidx)
    pltpu.sync_copy(edges_hbm.at[1, wid, pl.ds(0, IB)], didx)
    for b in range(NG):
        pltpu.async_copy(x_hbm.at[sidx.at[b]], rows[b], gsem[b])

    plsc.subcore_barrier()

    for kb in range(NIB):
        if kb > 0:
            pltpu.sync_copy(edges_hbm.at[0, wid, pl.ds(kb * IB, IB)], sidx)
            pltpu.sync_copy(edges_hbm.at[1, wid, pl.ds(kb * IB, IB)], didx)
            for b in range(NG):
                pltpu.async_copy(x_hbm.at[sidx.at[b]], rows[b], gsem[b])

        def group(t, _):
            for b in range(NB):
                jj = t * NB + b
                br = (b + NG) % NB

                @pl.when(jj >= NSC)
                def _recycle():
                    pltpu.make_async_copy(
                        rows[br], acc_sh.at[didx.at[jj - NSC]], ssem[br]
                    ).wait()

                @pl.when(jj + NG < IB)
                def _prefetch():
                    pltpu.async_copy(x_hbm.at[sidx.at[jj + NG]], rows[br], gsem[br])

                pltpu.make_async_copy(x_hbm.at[sidx.at[jj]], rows[b], gsem[b]).wait()
                pltpu.async_copy(rows[b], acc_sh.at[didx.at[jj]], ssem[b], add=True)
            return _

        lax.fori_loop(0, IB // NB, group, None)

        for k in range(NSC):
            jj = IB - NSC + k
            pltpu.make_async_copy(
                rows[jj % NB], acc_sh.at[didx.at[jj]], ssem[jj % NB]
            ).wait()

    plsc.subcore_barrier()

    pltpu.sync_copy(
        acc_sh.at[pl.ds(s * ROWS_PER_TILE, ROWS_PER_TILE)],
        out_hbm.at[c, pl.ds(s * ROWS_PER_TILE, ROWS_PER_TILE)],
    )


_RB = 2048


def _deg_of(deg_ref):
    return jnp.maximum(jnp.sum(deg_ref[...], axis=0), 1.0)


def _tc_prep_body(deg_ref, feat_ref, z0_ref):
    nrm = lax.rsqrt(_deg_of(deg_ref))[:, None]
    z0_ref[...] = feat_ref[...] * nrm


_tc_prep = pl.pallas_call(
    _tc_prep_body,
    grid=(NP // _RB,),
    in_specs=[
        pl.BlockSpec((NW, _RB), lambda i: (0, i)),
        pl.BlockSpec((_RB, D), lambda i: (i, 0)),
    ],
    out_specs=pl.BlockSpec((_RB, D), lambda i: (i, 0)),
    out_shape=jax.ShapeDtypeStruct((NP, D), jnp.float32),
)


def _tc_scale_body(acc_ref, deg_ref, z1_ref):
    inv = (1.0 / _deg_of(deg_ref))[:, None]
    z1_ref[...] = (acc_ref[0] + acc_ref[1]) * inv


_tc_scale = pl.pallas_call(
    _tc_scale_body,
    grid=(NP // _RB,),
    in_specs=[
        pl.BlockSpec((NC, _RB, D), lambda i: (0, i, 0)),
        pl.BlockSpec((NW, _RB), lambda i: (0, i)),
    ],
    out_specs=pl.BlockSpec((_RB, D), lambda i: (i, 0)),
    out_shape=jax.ShapeDtypeStruct((NP, D), jnp.float32),
)


def _tc_final_body(acc_ref, deg_ref, w_ref, b_ref, out_ref):
    nrm = lax.rsqrt(_deg_of(deg_ref))[:, None]
    h = (acc_ref[0] + acc_ref[1]) * nrm
    out_ref[...] = (
        jnp.dot(h, w_ref[...], preferred_element_type=jnp.float32) + b_ref[...]
    )


_tc_final = pl.pallas_call(
    _tc_final_body,
    grid=(NP // _RB,),
    in_specs=[
        pl.BlockSpec((NC, _RB, D), lambda i: (0, i, 0)),
        pl.BlockSpec((NW, _RB), lambda i: (0, i)),
        pl.BlockSpec((D, D), lambda i: (0, 0)),
        pl.BlockSpec((1, D), lambda i: (0, 0)),
    ],
    out_specs=pl.BlockSpec((_RB, D), lambda i: (i, 0)),
    out_shape=jax.ShapeDtypeStruct((NP, D), jnp.float32),
)


def kernel(feat, edge_index, weight, bias):
    feat_p = jnp.pad(feat, ((0, NP - N), (0, 0)))
    pad_idx = (jnp.arange(EPAD - E, dtype=jnp.int32) % (NP - N)) + N
    edges_p = jnp.concatenate([edge_index, jnp.stack([pad_idx, pad_idx])], axis=1)
    edges_r = edges_p.reshape(2, NW, NCH, CH)

    deg_parts = _sc_degree(edges_r)
    z0 = _tc_prep(deg_parts, feat_p)
    acc_a = _sc_hop(z0, edges_r)
    z1 = _tc_scale(acc_a, deg_parts)
    acc_b = _sc_hop(z1, edges_r)
    out = _tc_final(acc_b, deg_parts, weight, bias.reshape(1, D))
    return out[:N]
